# baseline (speedup 1.0000x reference)
import math

import jax
import jax.numpy as jnp
from jax.experimental import pallas as pl
from jax.experimental.pallas import tpu as pltpu

_K = 4
_LOG_F = math.log(128.0)


def _ln_rows(x, g, b, eps=1e-5):
    m = jnp.mean(x, axis=-1, keepdims=True)
    v = jnp.mean((x - m) ** 2, axis=-1, keepdims=True)
    return (x - m) * jax.lax.rsqrt(v + eps) * g + b


def _gelu(x):
    return 0.5 * x * (1.0 + jax.lax.erf(x * (1.0 / math.sqrt(2.0))))


def _fused_kernel(x_ref, ln1g_ref, ln1b_ref, noopw_ref, noopb_ref,
                  cw1_ref, cb1_ref, cw2_ref, cb2_ref, cw3_ref, cb3_ref,
                  emb_ref, ln2g_ref, ln2b_ref,
                  fw1_ref, fb1_ref, fw2_ref, fb2_ref, fw3_ref, fb3_ref,
                  out_init_ref,
                  out_ref,
                  fw1a16_ref, fw1b16_ref, emb16_ref, fw2h_ref, fw3h_ref,
                  noop_s, coarse_s, idx_s, upd_s):
    f32 = jnp.float32
    bf16 = jnp.bfloat16
    i = pl.program_id(0)
    j = pl.program_id(1)
    C = cw1_ref.shape[0]
    ntot = cw3_ref.shape[1]
    ftot = fw3_ref.shape[1]

    @pl.when((i == 0) & (j == 0))
    def _cast_weights():
        fw1a16_ref[...] = fw1_ref[0:C, :].astype(bf16)
        fw1b16_ref[...] = fw1_ref[C:2 * C, :].astype(bf16)
        emb16_ref[...] = emb_ref[...].astype(bf16)
        fw2h_ref[...] = fw2_ref[...].astype(bf16)
        fw3h_ref[...] = fw3_ref[...].astype(bf16)

    @pl.when(j == 0)
    def _coarse_and_fine():
        x = x_ref[...]
        bt = x.shape[0]
        xln = _ln_rows(x, ln1g_ref[...], ln1b_ref[...])
        noop_s[...] = (jnp.dot(xln, noopw_ref[...],
                               preferred_element_type=f32) + noopb_ref[...])
        h = _gelu(jnp.dot(xln, cw1_ref[...], preferred_element_type=f32)
                  + cb1_ref[...])
        h = _gelu(jnp.dot(h, cw2_ref[...], preferred_element_type=f32)
                  + cb2_ref[...])
        coarse = (jnp.dot(h, cw3_ref[...], preferred_element_type=f32)
                  + cb3_ref[...])
        coarse_s[...] = coarse

        iota_n = jax.lax.broadcasted_iota(jnp.int32, (bt, ntot), 1)
        vals = coarse
        idxs = []
        for k in range(_K):
            m = jnp.max(vals, axis=-1, keepdims=True)
            idxk = jnp.min(jnp.where(vals == m, iota_n, ntot), axis=-1,
                           keepdims=True)
            idxs.append(idxk)
            idx_s[:, k:k + 1] = idxk
            vals = jnp.where(iota_n == idxk, jnp.float32(-jnp.inf), vals)

        t = jnp.dot(xln.astype(bf16), fw1a16_ref[...],
                    preferred_element_type=f32)
        for k in range(_K):
            oh = (iota_n == idxs[k]).astype(bf16)
            e = jnp.dot(oh, emb16_ref[...], preferred_element_type=f32)
            e = _ln_rows(e, ln2g_ref[...], ln2b_ref[...])
            hf = _gelu(t + jnp.dot(e.astype(bf16), fw1b16_ref[...],
                                   preferred_element_type=f32)
                       + fb1_ref[...])
            hf = _gelu(jnp.dot(hf.astype(bf16), fw2h_ref[...],
                               preferred_element_type=f32)
                       + fb2_ref[...])
            f = (jnp.dot(hf.astype(bf16), fw3h_ref[...],
                         preferred_element_type=f32)
                 + fb3_ref[...])
            m = jnp.max(f, axis=-1, keepdims=True)
            lse = m + jnp.log(jnp.sum(jnp.exp(f - m), axis=-1,
                                      keepdims=True))
            upd_s[:, k * ftot:(k + 1) * ftot] = f + _LOG_F - lse

    bt = out_ref.shape[0]
    rows = pl.ds(j * bt, bt)
    coarse = coarse_s[rows, :]
    upd = upd_s[rows, :]
    noop = noop_s[rows, :]
    jj = jax.lax.broadcasted_iota(jnp.int32, (ftot, 2048), 0)
    mm = jax.lax.broadcasted_iota(jnp.int32, (ftot, 2048), 1)
    T = (jj == 16 * (mm // 256) + mm % 16).astype(f32)
    c16 = jax.lax.broadcasted_iota(jnp.int32, (16, 2048), 0)
    m16 = jax.lax.broadcasted_iota(jnp.int32, (16, 2048), 1)
    M16 = (c16 == (m16 // 16) % 16).astype(f32)
    M16h = (c16 == (m16 // 16) % 16).astype(bf16)

    updbig = [jnp.dot(upd[:, 128 * k:128 * (k + 1)], T,
                      preferred_element_type=f32) for k in range(_K)]
    iota_e = jax.lax.broadcasted_iota(jnp.int32, (bt, ntot), 1)
    idx_e = idx_s[rows, :]
    sstack = jnp.concatenate(
        [(iota_e == idx_e[:, k:k + 1]).astype(bf16) for k in range(_K)],
        axis=0)

    pieces = [noop]
    for ch in range(16):
        seg = jnp.dot(coarse[:, 16 * ch:16 * (ch + 1)], M16,
                      preferred_element_type=f32) - _LOG_F
        sexp = jnp.dot(sstack[:, 16 * ch:16 * (ch + 1)], M16h,
                       preferred_element_type=f32)
        for k in range(_K):
            seg = seg + sexp[k * bt:(k + 1) * bt] * updbig[k]
        pieces.append(seg)
    out_ref[...] = jnp.concatenate(pieces, axis=-1)


def _full(w):
    return pl.BlockSpec(w.shape, lambda i, j: (0,) * w.ndim)


def kernel(x, ln1_g, ln1_b, noop_W, noop_b, cW1, cb1, cW2, cb2, cW3, cb3,
           emb, ln2_g, ln2_b, fW1, fb1, fW2, fb2, fW3, fb3):
    B, C = x.shape
    NTOT = cW3.shape[1]
    FTOT = fW3.shape[1]
    f32 = jnp.float32
    bf16 = jnp.bfloat16

    BT = 256
    NJ = 4
    bte = BT // NJ

    out_init = jnp.zeros((B, 1 + NTOT * FTOT), f32)
    ins = (x, ln1_g, ln1_b, noop_W, noop_b,
           cW1, cb1, cW2, cb2, cW3, cb3,
           emb, ln2_g, ln2_b, fW1, fb1, fW2, fb2, fW3, fb3, out_init)
    out = pl.pallas_call(
        _fused_kernel,
        grid=(B // BT, NJ),
        in_specs=[pl.BlockSpec((BT, C), lambda i, j: (i, 0))]
        + [_full(v) for v in ins[1:-1]]
        + [pl.BlockSpec(memory_space=pl.ANY)],
        out_specs=pl.BlockSpec((bte, 1 + NTOT * FTOT),
                               lambda i, j: (i * NJ + j, 0)),
        out_shape=jax.ShapeDtypeStruct((B, 1 + NTOT * FTOT), f32),
        input_output_aliases={len(ins) - 1: 0},
        scratch_shapes=[
            pltpu.VMEM((C, C), bf16),
            pltpu.VMEM((C, C), bf16),
            pltpu.VMEM((NTOT, C), bf16),
            pltpu.VMEM((C, C), bf16),
            pltpu.VMEM((C, FTOT), bf16),
            pltpu.VMEM((BT, 1), f32),
            pltpu.VMEM((BT, NTOT), f32),
            pltpu.VMEM((BT, _K), jnp.int32),
            pltpu.VMEM((BT, _K * FTOT), f32),
        ],
    )(*ins)
    return out


# speedup vs baseline: 1.1510x; 1.1510x over previous
import math

import jax
import jax.numpy as jnp
from jax.experimental import pallas as pl
from jax.experimental.pallas import tpu as pltpu

_K = 4
_LOG_F = math.log(128.0)


def _ln_rows(x, g, b, eps=1e-5):
    m = jnp.mean(x, axis=-1, keepdims=True)
    v = jnp.mean((x - m) ** 2, axis=-1, keepdims=True)
    return (x - m) * jax.lax.rsqrt(v + eps) * g + b


def _gelu(x):
    return 0.5 * x * (1.0 + jax.lax.erf(x * (1.0 / math.sqrt(2.0))))


def _fused_kernel(x_ref, ln1g_ref, ln1b_ref, noopw_ref, noopb_ref,
                  cw1_ref, cb1_ref, cw2_ref, cb2_ref, cw3_ref, cb3_ref,
                  emb_ref, ln2g_ref, ln2b_ref,
                  fw1_ref, fb1_ref, fw2_ref, fb2_ref, fw3_ref, fb3_ref,
                  out_ref,
                  fw1a16_ref, fw1b16_ref, emb16_ref, fw2h_ref, fw3h_ref,
                  noop_s, coarse_s, idx_s, upd_s):
    f32 = jnp.float32
    bf16 = jnp.bfloat16
    i = pl.program_id(0)
    j = pl.program_id(1)
    C = cw1_ref.shape[0]
    ntot = cw3_ref.shape[1]
    ftot = fw3_ref.shape[1]

    @pl.when((i == 0) & (j == 0))
    def _cast_weights():
        fw1a16_ref[...] = fw1_ref[0:C, :].astype(bf16)
        fw1b16_ref[...] = fw1_ref[C:2 * C, :].astype(bf16)
        emb16_ref[...] = emb_ref[...].astype(bf16)
        fw2h_ref[...] = fw2_ref[...].astype(bf16)
        fw3h_ref[...] = fw3_ref[...].astype(bf16)

    @pl.when(j == 0)
    def _coarse_and_fine():
        x = x_ref[...]
        bt = x.shape[0]
        xln = _ln_rows(x, ln1g_ref[...], ln1b_ref[...])
        noop_s[...] = (jnp.dot(xln, noopw_ref[...],
                               preferred_element_type=f32) + noopb_ref[...])
        h = _gelu(jnp.dot(xln, cw1_ref[...], preferred_element_type=f32)
                  + cb1_ref[...])
        h = _gelu(jnp.dot(h, cw2_ref[...], preferred_element_type=f32)
                  + cb2_ref[...])
        coarse = (jnp.dot(h, cw3_ref[...], preferred_element_type=f32)
                  + cb3_ref[...])
        coarse_s[...] = coarse

        iota_n = jax.lax.broadcasted_iota(jnp.int32, (bt, ntot), 1)
        vals = coarse
        idxs = []
        for k in range(_K):
            m = jnp.max(vals, axis=-1, keepdims=True)
            idxk = jnp.min(jnp.where(vals == m, iota_n, ntot), axis=-1,
                           keepdims=True)
            idxs.append(idxk)
            idx_s[:, k:k + 1] = idxk
            vals = jnp.where(iota_n == idxk, jnp.float32(-jnp.inf), vals)

        t = jnp.dot(xln.astype(bf16), fw1a16_ref[...],
                    preferred_element_type=f32)
        for k in range(_K):
            oh = (iota_n == idxs[k]).astype(bf16)
            e = jnp.dot(oh, emb16_ref[...], preferred_element_type=f32)
            e = _ln_rows(e, ln2g_ref[...], ln2b_ref[...])
            hf = _gelu(t + jnp.dot(e.astype(bf16), fw1b16_ref[...],
                                   preferred_element_type=f32)
                       + fb1_ref[...])
            hf = _gelu(jnp.dot(hf.astype(bf16), fw2h_ref[...],
                               preferred_element_type=f32)
                       + fb2_ref[...])
            f = (jnp.dot(hf.astype(bf16), fw3h_ref[...],
                         preferred_element_type=f32)
                 + fb3_ref[...])
            m = jnp.max(f, axis=-1, keepdims=True)
            lse = m + jnp.log(jnp.sum(jnp.exp(f - m), axis=-1,
                                      keepdims=True))
            upd_s[:, k * ftot:(k + 1) * ftot] = f + _LOG_F - lse

    bt = out_ref.shape[0]
    rows = pl.ds(j * bt, bt)
    coarse = coarse_s[rows, :]
    upd = upd_s[rows, :]
    noop = noop_s[rows, :]
    jj = jax.lax.broadcasted_iota(jnp.int32, (ftot, 2048), 0)
    mm = jax.lax.broadcasted_iota(jnp.int32, (ftot, 2048), 1)
    T = (jj == 16 * (mm // 256) + mm % 16).astype(f32)
    c16 = jax.lax.broadcasted_iota(jnp.int32, (16, 2048), 0)
    m16 = jax.lax.broadcasted_iota(jnp.int32, (16, 2048), 1)
    M16 = (c16 == (m16 // 16) % 16).astype(f32)
    M16h = (c16 == (m16 // 16) % 16).astype(bf16)

    updbig = [jnp.dot(upd[:, 128 * k:128 * (k + 1)], T,
                      preferred_element_type=f32) for k in range(_K)]
    iota_e = jax.lax.broadcasted_iota(jnp.int32, (bt, ntot), 1)
    idx_e = idx_s[rows, :]
    sstack = jnp.concatenate(
        [(iota_e == idx_e[:, k:k + 1]).astype(bf16) for k in range(_K)],
        axis=0)

    pieces = [noop]
    for ch in range(16):
        seg = jnp.dot(coarse[:, 16 * ch:16 * (ch + 1)], M16,
                      preferred_element_type=f32) - _LOG_F
        sexp = jnp.dot(sstack[:, 16 * ch:16 * (ch + 1)], M16h,
                       preferred_element_type=f32)
        for k in range(_K):
            seg = seg + sexp[k * bt:(k + 1) * bt] * updbig[k]
        pieces.append(seg)
    out_ref[...] = jnp.concatenate(pieces, axis=-1)


def _full(w):
    return pl.BlockSpec(w.shape, lambda i, j: (0,) * w.ndim)


def kernel(x, ln1_g, ln1_b, noop_W, noop_b, cW1, cb1, cW2, cb2, cW3, cb3,
           emb, ln2_g, ln2_b, fW1, fb1, fW2, fb2, fW3, fb3):
    B, C = x.shape
    NTOT = cW3.shape[1]
    FTOT = fW3.shape[1]
    f32 = jnp.float32
    bf16 = jnp.bfloat16

    BT = 256
    NJ = 4
    bte = BT // NJ

    ins = (x, ln1_g, ln1_b, noop_W, noop_b,
           cW1, cb1, cW2, cb2, cW3, cb3,
           emb, ln2_g, ln2_b, fW1, fb1, fW2, fb2, fW3, fb3)
    out = pl.pallas_call(
        _fused_kernel,
        grid=(B // BT, NJ),
        in_specs=[pl.BlockSpec((BT, C), lambda i, j: (i, 0))]
        + [_full(v) for v in ins[1:]],
        out_specs=pl.BlockSpec((bte, 1 + NTOT * FTOT),
                               lambda i, j: (i * NJ + j, 0)),
        out_shape=jax.ShapeDtypeStruct((B, 1 + NTOT * FTOT), f32),
        scratch_shapes=[
            pltpu.VMEM((C, C), bf16),
            pltpu.VMEM((C, C), bf16),
            pltpu.VMEM((NTOT, C), bf16),
            pltpu.VMEM((C, C), bf16),
            pltpu.VMEM((C, FTOT), bf16),
            pltpu.VMEM((BT, 1), f32),
            pltpu.VMEM((BT, NTOT), f32),
            pltpu.VMEM((BT, _K), jnp.int32),
            pltpu.VMEM((BT, _K * FTOT), f32),
        ],
    )(*ins)
    return out


# speedup vs baseline: 1.1518x; 1.0007x over previous
import math

import jax
import jax.numpy as jnp
from jax.experimental import pallas as pl
from jax.experimental.pallas import tpu as pltpu

_K = 4
_LOG_F = math.log(128.0)


def _ln_rows(x, g, b, eps=1e-5):
    m = jnp.mean(x, axis=-1, keepdims=True)
    v = jnp.mean((x - m) ** 2, axis=-1, keepdims=True)
    return (x - m) * jax.lax.rsqrt(v + eps) * g + b


def _gelu(x):
    return 0.5 * x * (1.0 + jax.lax.erf(x * (1.0 / math.sqrt(2.0))))


def _fused_kernel(x_ref, ln1g_ref, ln1b_ref, noopw_ref, noopb_ref,
                  cw1_ref, cb1_ref, cw2_ref, cb2_ref, cw3_ref, cb3_ref,
                  emb_ref, ln2g_ref, ln2b_ref,
                  fw1_ref, fb1_ref, fw2_ref, fb2_ref, fw3_ref, fb3_ref,
                  out_ref,
                  fw1a16_ref, fw1b16_ref, emb16_ref, fw2h_ref, fw3h_ref,
                  noop_s, coarse_s, idx_s, upd_s):
    f32 = jnp.float32
    bf16 = jnp.bfloat16
    i = pl.program_id(0)
    j = pl.program_id(1)
    C = cw1_ref.shape[0]
    ntot = cw3_ref.shape[1]
    ftot = fw3_ref.shape[1]

    @pl.when((i == 0) & (j == 0))
    def _cast_weights():
        fw1a16_ref[...] = fw1_ref[0:C, :].astype(bf16)
        fw1b16_ref[...] = fw1_ref[C:2 * C, :].astype(bf16)
        emb16_ref[...] = emb_ref[...].astype(bf16)
        fw2h_ref[...] = fw2_ref[...].astype(bf16)
        fw3h_ref[...] = fw3_ref[...].astype(bf16)

    @pl.when(j == 0)
    def _coarse_and_fine():
        x = x_ref[...]
        bt = x.shape[0]
        xln = _ln_rows(x, ln1g_ref[...], ln1b_ref[...])
        noop_s[...] = (jnp.dot(xln, noopw_ref[...],
                               preferred_element_type=f32) + noopb_ref[...])
        h = _gelu(jnp.dot(xln, cw1_ref[...], preferred_element_type=f32)
                  + cb1_ref[...])
        h = _gelu(jnp.dot(h, cw2_ref[...], preferred_element_type=f32)
                  + cb2_ref[...])
        coarse = (jnp.dot(h, cw3_ref[...], preferred_element_type=f32)
                  + cb3_ref[...])
        coarse_s[...] = coarse

        iota_n = jax.lax.broadcasted_iota(jnp.int32, (bt, ntot), 1)
        vals = coarse
        idxs = []
        for k in range(_K):
            m = jnp.max(vals, axis=-1, keepdims=True)
            idxk = jnp.min(jnp.where(vals == m, iota_n, ntot), axis=-1,
                           keepdims=True)
            idxs.append(idxk)
            idx_s[:, k:k + 1] = idxk
            vals = jnp.where(iota_n == idxk, jnp.float32(-jnp.inf), vals)

        t = jnp.dot(xln.astype(bf16), fw1a16_ref[...],
                    preferred_element_type=f32)
        for k in range(_K):
            oh = (iota_n == idxs[k]).astype(bf16)
            e = jnp.dot(oh, emb16_ref[...], preferred_element_type=f32)
            e = _ln_rows(e, ln2g_ref[...], ln2b_ref[...])
            hf = _gelu(t + jnp.dot(e.astype(bf16), fw1b16_ref[...],
                                   preferred_element_type=f32)
                       + fb1_ref[...])
            hf = _gelu(jnp.dot(hf.astype(bf16), fw2h_ref[...],
                               preferred_element_type=f32)
                       + fb2_ref[...])
            f = (jnp.dot(hf.astype(bf16), fw3h_ref[...],
                         preferred_element_type=f32)
                 + fb3_ref[...])
            m = jnp.max(f, axis=-1, keepdims=True)
            lse = m + jnp.log(jnp.sum(jnp.exp(f - m), axis=-1,
                                      keepdims=True))
            upd_s[:, k * ftot:(k + 1) * ftot] = f + _LOG_F - lse

    bt = out_ref.shape[0]
    rows = pl.ds(j * bt, bt)
    coarse = coarse_s[rows, :]
    upd = upd_s[rows, :]
    noop = noop_s[rows, :]
    jj = jax.lax.broadcasted_iota(jnp.int32, (ftot, 2048), 0)
    mm = jax.lax.broadcasted_iota(jnp.int32, (ftot, 2048), 1)
    T = (jj == 16 * (mm // 256) + mm % 16).astype(bf16)
    c16 = jax.lax.broadcasted_iota(jnp.int32, (16, 2048), 0)
    m16 = jax.lax.broadcasted_iota(jnp.int32, (16, 2048), 1)
    M16 = (c16 == (m16 // 16) % 16).astype(f32)
    M16h = (c16 == (m16 // 16) % 16).astype(bf16)

    updbig = [jnp.dot(upd[:, 128 * k:128 * (k + 1)].astype(bf16), T,
                      preferred_element_type=f32) for k in range(_K)]
    iota_e = jax.lax.broadcasted_iota(jnp.int32, (bt, ntot), 1)
    idx_e = idx_s[rows, :]
    sstack = jnp.concatenate(
        [(iota_e == idx_e[:, k:k + 1]).astype(bf16) for k in range(_K)],
        axis=0)

    pieces = [noop]
    for ch in range(16):
        seg = jnp.dot(coarse[:, 16 * ch:16 * (ch + 1)], M16,
                      preferred_element_type=f32) - _LOG_F
        sexp = jnp.dot(sstack[:, 16 * ch:16 * (ch + 1)], M16h,
                       preferred_element_type=f32)
        for k in range(_K):
            seg = seg + sexp[k * bt:(k + 1) * bt] * updbig[k]
        pieces.append(seg)
    out_ref[...] = jnp.concatenate(pieces, axis=-1)


def _full(w):
    return pl.BlockSpec(w.shape, lambda i, j: (0,) * w.ndim)


def kernel(x, ln1_g, ln1_b, noop_W, noop_b, cW1, cb1, cW2, cb2, cW3, cb3,
           emb, ln2_g, ln2_b, fW1, fb1, fW2, fb2, fW3, fb3):
    B, C = x.shape
    NTOT = cW3.shape[1]
    FTOT = fW3.shape[1]
    f32 = jnp.float32
    bf16 = jnp.bfloat16

    BT = 256
    NJ = 4
    bte = BT // NJ

    ins = (x, ln1_g, ln1_b, noop_W, noop_b,
           cW1, cb1, cW2, cb2, cW3, cb3,
           emb, ln2_g, ln2_b, fW1, fb1, fW2, fb2, fW3, fb3)
    out = pl.pallas_call(
        _fused_kernel,
        grid=(B // BT, NJ),
        in_specs=[pl.BlockSpec((BT, C), lambda i, j: (i, 0))]
        + [_full(v) for v in ins[1:]],
        out_specs=pl.BlockSpec((bte, 1 + NTOT * FTOT),
                               lambda i, j: (i * NJ + j, 0)),
        out_shape=jax.ShapeDtypeStruct((B, 1 + NTOT * FTOT), f32),
        scratch_shapes=[
            pltpu.VMEM((C, C), bf16),
            pltpu.VMEM((C, C), bf16),
            pltpu.VMEM((NTOT, C), bf16),
            pltpu.VMEM((C, C), bf16),
            pltpu.VMEM((C, FTOT), bf16),
            pltpu.VMEM((BT, 1), f32),
            pltpu.VMEM((BT, NTOT), f32),
            pltpu.VMEM((BT, _K), jnp.int32),
            pltpu.VMEM((BT, _K * FTOT), f32),
        ],
    )(*ins)
    return out
